# baseline (speedup 1.0000x reference)
import functools

import jax
import jax.numpy as jnp
from jax import lax
from jax.experimental import pallas as pl
from jax.experimental.pallas import tpu as pltpu
from jax.experimental.pallas import tpu_sc as plsc

NC = 2
NS = 16
NW = NC * NS
CK = 120
NSLOT = 3
NEB = 6
UNROLL = 6


def _sc_mesh():
    return plsc.VectorSubcoreMesh(
        core_axis_name="c", subcore_axis_name="s", num_cores=NC,
        num_subcores=NS)


def _zero_slab(rows, acc, s, slab, nsub, nrow):
    def zr(e, carry):
        for f in range(nsub):
            rows[e, pl.ds(f * 16, 16)] = jnp.zeros((16,), jnp.float32)
        return carry
    lax.fori_loop(0, nrow, zr, 0)
    done = 0
    while done < slab:
        step = min(nrow, slab - done)
        pltpu.sync_copy(rows.at[pl.ds(0, step)],
                        acc.at[pl.ds(s * slab + done, step)])
        done += step


def _deg_body(npad, nchunks, earr_hbm, ew_hbm, degp_hbm, *scr):
    ebufs = scr[0:NEB]
    ewbs = scr[NEB:2 * NEB]
    zbuf, acc, sem_i, sem_s = scr[2 * NEB:]
    slab = npad // NS
    c = lax.axis_index("c")
    s = lax.axis_index("s")
    w = c * NS + s
    nk = nchunks // NW
    niter = nk // UNROLL

    def zb(j, carry):
        zbuf[pl.ds(j * 16, 16)] = jnp.zeros((16,), jnp.float32)
        return carry
    lax.fori_loop(0, (slab + 15) // 16, zb, 0)
    pltpu.sync_copy(zbuf.at[pl.ds(0, slab)], acc.at[pl.ds(s * slab, slab)])
    plsc.subcore_barrier()

    def start_idx(slot, ch):
        pltpu.async_copy(earr_hbm.at[ch], ebufs[slot], sem_i)
        pltpu.async_copy(ew_hbm.at[ch], ewbs[slot], sem_i)

    def wait_idx(slot):
        pltpu.make_async_copy(earr_hbm.at[0], ebufs[slot], sem_i).wait()
        pltpu.make_async_copy(ew_hbm.at[0], ewbs[slot], sem_i).wait()

    def start_scatter(slot):
        pltpu.async_copy(ewbs[slot], acc.at[ebufs[slot].at[1]], sem_s,
                         add=True)

    def wait_scatter(slot):
        pltpu.make_async_copy(ewbs[slot], acc.at[ebufs[slot].at[1]],
                              sem_s).wait()

    for j in range(4):
        start_idx(j, w + NW * j)
    wait_idx(0)
    wait_idx(1)

    def it_body(it, carry):
        for u in range(UNROLL):
            if u >= 4:
                pl.when(it < niter - 1)(lambda: wait_idx((u + 2) % NEB))
            else:
                wait_idx((u + 2) % NEB)
            if u < 2:
                pl.when(it > 0)(lambda: wait_scatter((u + 2) % NEB))
            else:
                wait_scatter((u + 2) % NEB)
            start_scatter(u)
            ch4 = w + NW * (UNROLL * it + u + 4)
            if u >= 2:
                pl.when(it < niter - 1)(lambda: start_idx((u + 4) % NEB, ch4))
            else:
                start_idx((u + 4) % NEB, ch4)
        return carry

    lax.fori_loop(0, niter, it_body, 0)
    for _ in range(2):
        wait_scatter(0)
    plsc.subcore_barrier()
    pltpu.sync_copy(acc.at[pl.ds(s * slab, slab)],
                    degp_hbm.at[c, pl.ds(s * slab, slab)])


def _deg_call(earr, ew2p, npad):
    nchunks = earr.shape[0]
    body = functools.partial(_deg_body, npad, nchunks)
    return pl.kernel(
        body,
        out_type=jax.ShapeDtypeStruct((NC, npad), jnp.float32),
        mesh=_sc_mesh(),
        scratch_types=(
            [pltpu.VMEM((2, CK), jnp.int32) for _ in range(NEB)]
            + [pltpu.VMEM((CK,), jnp.float32) for _ in range(NEB)]
            + [
                pltpu.VMEM((((npad // NS + 15) // 16) * 16,), jnp.float32),
                pltpu.VMEM_SHARED((npad,), jnp.float32),
                pltpu.SemaphoreType.DMA,
                pltpu.SemaphoreType.DMA,
            ]
        ),
    )(earr, ew2p)


def _conv_body(npad, nchunks, d, earr_hbm, ew_hbm, tab_hbm, part_hbm, *scr):
    ebufs = scr[0:NEB]
    ewbs = scr[NEB:2 * NEB]
    rows, acc, sem_i, sem_g, sem_s = scr[2 * NEB:]
    slab = npad // NS
    c = lax.axis_index("c")
    s = lax.axis_index("s")
    w = c * NS + s
    nsub = d // 16
    nk = nchunks // NW
    niter = nk // UNROLL

    _zero_slab(rows, acc, s, slab, nsub, CK)
    plsc.subcore_barrier()

    def start_idx(slot, ch):
        pltpu.async_copy(earr_hbm.at[ch], ebufs[slot], sem_i)
        pltpu.async_copy(ew_hbm.at[ch], ewbs[slot], sem_i)

    def wait_idx(slot):
        pltpu.make_async_copy(earr_hbm.at[0], ebufs[slot], sem_i).wait()
        pltpu.make_async_copy(ew_hbm.at[0], ewbs[slot], sem_i).wait()

    def start_gather(kslot, eslot):
        pltpu.async_copy(tab_hbm.at[ebufs[eslot].at[0]],
                         rows.at[pl.ds(kslot * CK, CK)], sem_g)

    def wait_gather(kslot, eslot):
        pltpu.make_async_copy(tab_hbm.at[ebufs[eslot].at[0]],
                              rows.at[pl.ds(kslot * CK, CK)], sem_g).wait()

    def start_scatter(kslot, eslot):
        pltpu.async_copy(rows.at[pl.ds(kslot * CK, CK)],
                         acc.at[ebufs[eslot].at[1]], sem_s, add=True)

    def wait_scatter(kslot, eslot):
        pltpu.make_async_copy(rows.at[pl.ds(kslot * CK, CK)],
                              acc.at[ebufs[eslot].at[1]], sem_s).wait()

    def scale(kslot, eslot):
        def grp(l16, carry2):
            ew16 = ewbs[eslot][pl.ds(l16 * 16, 16)]
            for i in range(16):
                sc = ew16[i]
                e = kslot * CK + l16 * 16 + i
                for f in range(nsub):
                    v = rows[e, pl.ds(f * 16, 16)]
                    rows[e, pl.ds(f * 16, 16)] = v * sc
            return carry2
        lax.fori_loop(0, CK // 16, grp, 0)
        if CK % 16:
            ew16 = ewbs[eslot][pl.ds(CK - 16, 16)]
            for i in range(16 - CK % 16, 16):
                sc = ew16[i]
                e = kslot * CK + CK - 16 + i
                for f in range(nsub):
                    v = rows[e, pl.ds(f * 16, 16)]
                    rows[e, pl.ds(f * 16, 16)] = v * sc

    for j in range(4):
        start_idx(j, w + NW * j)
    wait_idx(0)
    start_gather(0, 0)
    wait_idx(1)
    start_gather(1, 1)

    def it_body(it, carry):
        for u in range(UNROLL):
            if u >= 4:
                pl.when(it < niter - 1)(lambda: wait_idx((u + 2) % NEB))
            else:
                wait_idx((u + 2) % NEB)
            if u == 0:
                pl.when(it > 0)(
                    lambda: wait_scatter((u + 2) % NSLOT, (u + 2) % NEB))
            else:
                wait_scatter((u + 2) % NSLOT, (u + 2) % NEB)
            if u >= 4:
                pl.when(it < niter - 1)(
                    lambda: start_gather((u + 2) % NSLOT, (u + 2) % NEB))
            else:
                start_gather((u + 2) % NSLOT, (u + 2) % NEB)

            wait_gather(u % NSLOT, u)

            ch4 = w + NW * (UNROLL * it + u + 4)
            if u >= 2:
                pl.when(it < niter - 1)(lambda: start_idx((u + 4) % NEB, ch4))
            else:
                start_idx((u + 4) % NEB, ch4)

            scale(u % NSLOT, u)
            start_scatter(u % NSLOT, u)
        return carry

    lax.fori_loop(0, niter, it_body, 0)
    wait_scatter(0, 0)
    plsc.subcore_barrier()
    pltpu.sync_copy(acc.at[pl.ds(s * slab, slab)],
                    part_hbm.at[c, pl.ds(s * slab, slab)])


def _conv_call(earr, ew2p, table):
    nchunks = earr.shape[0]
    d = table.shape[1]
    n = table.shape[0]
    npad = ((n + 127) // 128) * 128
    body = functools.partial(_conv_body, npad, nchunks, d)
    return pl.kernel(
        body,
        out_type=jax.ShapeDtypeStruct((NC, npad, d), jnp.float32),
        mesh=_sc_mesh(),
        scratch_types=(
            [pltpu.VMEM((2, CK), jnp.int32) for _ in range(NEB)]
            + [pltpu.VMEM((CK,), jnp.float32) for _ in range(NEB)]
            + [
                pltpu.VMEM((NSLOT * CK, d), jnp.float32),
                pltpu.VMEM_SHARED((npad, d), jnp.float32),
                pltpu.SemaphoreType.DMA,
                pltpu.SemaphoreType.DMA,
                pltpu.SemaphoreType.DMA,
            ]
        ),
    )(earr, ew2p, table)


def _dinv(degp_ref, n):
    deg = degp_ref[0, :n] + degp_ref[1, :n] + 1.0
    return lax.rsqrt(deg).reshape(n, 1)


def _mm_body(x_ref, w_ref, out_ref):
    out_ref[...] = jnp.dot(x_ref[...], w_ref[...],
                           preferred_element_type=jnp.float32)


def _prescale_body(n, lin_ref, degp_ref, out_ref):
    dinv = _dinv(degp_ref, n)
    out_ref[...] = dinv * lin_ref[...]


def _mid_body(n, degp_ref, p_ref, linp_ref, w2_ref, b1_ref, out_ref):
    dinv = _dinv(degp_ref, n)
    ssum = p_ref[0, :n, :] + p_ref[1, :n, :] + linp_ref[...]
    h = jnp.maximum(dinv * ssum + b1_ref[...], 0.0)
    out_ref[...] = dinv * jnp.dot(h, w2_ref[...],
                                  preferred_element_type=jnp.float32)


def _final_body(n, degp_ref, p_ref, linp_ref, b2_ref, g_ref, be_ref, x_ref,
                out_ref):
    dinv = _dinv(degp_ref, n)
    out2 = dinv * (p_ref[0, :n, :] + p_ref[1, :n, :] + linp_ref[...]) \
        + b2_ref[...]
    mean = jnp.mean(out2, axis=0)
    var = jnp.mean((out2 - mean) ** 2, axis=0)
    y = g_ref[...] * (out2 - mean) * lax.rsqrt(var + 1e-5) + be_ref[...] \
        + x_ref[...]
    out_ref[...] = jnp.maximum(y, 0.0)


def _tc_call(body, out_shape, *args):
    return pl.pallas_call(
        body, out_shape=jax.ShapeDtypeStruct(out_shape, jnp.float32))(*args)


def kernel(x, edge_index, edge_weight, W1, b1, W2, b2, gamma, beta):
    n, d = x.shape
    e = edge_weight.shape[0]
    npad_deg = ((n + 16 * NS - 1) // (16 * NS)) * (16 * NS)
    assert d % 16 == 0 and n % NS == 0

    grp = CK * NW * UNROLL
    epad = ((e + grp - 1) // grp) * grp
    pad = epad - e
    ar = jnp.arange(pad, dtype=jnp.int32) % n
    row2p = jnp.concatenate([edge_index[0], ar]).reshape(-1, CK)
    col2p = jnp.concatenate([edge_index[1], ar]).reshape(-1, CK)
    ew2p = jnp.concatenate(
        [edge_weight, jnp.zeros((pad,), jnp.float32)]).reshape(-1, CK)
    earr = jnp.stack([row2p, col2p], axis=1)

    lin1 = _tc_call(_mm_body, (n, d), x, W1)
    degp = _deg_call(earr, ew2p, npad_deg)
    lin1p = _tc_call(functools.partial(_prescale_body, n), (n, d),
                     lin1, degp)
    part1 = _conv_call(earr, ew2p, lin1p)
    lin2p = _tc_call(functools.partial(_mid_body, n), (n, d),
                     degp, part1, lin1p, W2, b1)
    part2 = _conv_call(earr, ew2p, lin2p)
    out = _tc_call(functools.partial(_final_body, n), (n, d),
                   degp, part2, lin2p, b2, gamma, beta, x)
    return out


# speedup vs baseline: 1.0141x; 1.0141x over previous
import functools

import jax
import jax.numpy as jnp
from jax import lax
from jax.experimental import pallas as pl
from jax.experimental.pallas import tpu as pltpu
from jax.experimental.pallas import tpu_sc as plsc

NC = 2
NS = 16
NW = NC * NS
CK = 80
NSLOT = 4
NEB = 8
UNROLL = 8


def _sc_mesh():
    return plsc.VectorSubcoreMesh(
        core_axis_name="c", subcore_axis_name="s", num_cores=NC,
        num_subcores=NS)


def _zero_slab(rows, acc, s, slab, nsub, nrow):
    def zr(e, carry):
        for f in range(nsub):
            rows[e, pl.ds(f * 16, 16)] = jnp.zeros((16,), jnp.float32)
        return carry
    lax.fori_loop(0, nrow, zr, 0)
    done = 0
    while done < slab:
        step = min(nrow, slab - done)
        pltpu.sync_copy(rows.at[pl.ds(0, step)],
                        acc.at[pl.ds(s * slab + done, step)])
        done += step


def _deg_body(npad, nchunks, earr_hbm, ew_hbm, degp_hbm, *scr):
    ebufs = scr[0:NEB]
    ewbs = scr[NEB:2 * NEB]
    zbuf, acc, sem_i, sem_s = scr[2 * NEB:]
    slab = npad // NS
    c = lax.axis_index("c")
    s = lax.axis_index("s")
    w = c * NS + s
    nk = nchunks // NW
    niter = nk // UNROLL

    def zb(j, carry):
        zbuf[pl.ds(j * 16, 16)] = jnp.zeros((16,), jnp.float32)
        return carry
    lax.fori_loop(0, (slab + 15) // 16, zb, 0)
    pltpu.sync_copy(zbuf.at[pl.ds(0, slab)], acc.at[pl.ds(s * slab, slab)])
    plsc.subcore_barrier()

    def start_idx(slot, ch):
        pltpu.async_copy(earr_hbm.at[ch], ebufs[slot], sem_i)
        pltpu.async_copy(ew_hbm.at[ch], ewbs[slot], sem_i)

    def wait_idx(slot):
        pltpu.make_async_copy(earr_hbm.at[0], ebufs[slot], sem_i).wait()
        pltpu.make_async_copy(ew_hbm.at[0], ewbs[slot], sem_i).wait()

    def start_scatter(slot):
        pltpu.async_copy(ewbs[slot], acc.at[ebufs[slot].at[1]], sem_s,
                         add=True)

    def wait_scatter(slot):
        pltpu.make_async_copy(ewbs[slot], acc.at[ebufs[slot].at[1]],
                              sem_s).wait()

    for j in range(4):
        start_idx(j, w + NW * j)
    wait_idx(0)
    wait_idx(1)

    def it_body(it, carry):
        for u in range(UNROLL):
            if u >= 6:
                pl.when(it < niter - 1)(lambda: wait_idx((u + 2) % NEB))
            else:
                wait_idx((u + 2) % NEB)
            if u < 2:
                pl.when(it > 0)(lambda: wait_scatter((u + 2) % NEB))
            else:
                wait_scatter((u + 2) % NEB)
            start_scatter(u)
            ch4 = w + NW * (UNROLL * it + u + 4)
            if u >= 4:
                pl.when(it < niter - 1)(lambda: start_idx((u + 4) % NEB, ch4))
            else:
                start_idx((u + 4) % NEB, ch4)
        return carry

    lax.fori_loop(0, niter, it_body, 0)
    for _ in range(2):
        wait_scatter(0)
    plsc.subcore_barrier()
    pltpu.sync_copy(acc.at[pl.ds(s * slab, slab)],
                    degp_hbm.at[c, pl.ds(s * slab, slab)])


def _deg_call(earr, ew2p, npad):
    nchunks = earr.shape[0]
    body = functools.partial(_deg_body, npad, nchunks)
    return pl.kernel(
        body,
        out_type=jax.ShapeDtypeStruct((NC, npad), jnp.float32),
        mesh=_sc_mesh(),
        scratch_types=(
            [pltpu.VMEM((2, CK), jnp.int32) for _ in range(NEB)]
            + [pltpu.VMEM((CK,), jnp.float32) for _ in range(NEB)]
            + [
                pltpu.VMEM((((npad // NS + 15) // 16) * 16,), jnp.float32),
                pltpu.VMEM_SHARED((npad,), jnp.float32),
                pltpu.SemaphoreType.DMA,
                pltpu.SemaphoreType.DMA,
            ]
        ),
    )(earr, ew2p)


def _conv_body(npad, nchunks, d, earr_hbm, ew_hbm, tab_hbm, part_hbm, *scr):
    ebufs = scr[0:NEB]
    ewbs = scr[NEB:2 * NEB]
    rows, acc, sem_i, sem_g, sem_s = scr[2 * NEB:]
    slab = npad // NS
    c = lax.axis_index("c")
    s = lax.axis_index("s")
    w = c * NS + s
    nsub = d // 16
    nk = nchunks // NW
    niter = nk // UNROLL

    _zero_slab(rows, acc, s, slab, nsub, CK)
    plsc.subcore_barrier()

    def start_idx(slot, ch):
        pltpu.async_copy(earr_hbm.at[ch], ebufs[slot], sem_i)
        pltpu.async_copy(ew_hbm.at[ch], ewbs[slot], sem_i)

    def wait_idx(slot):
        pltpu.make_async_copy(earr_hbm.at[0], ebufs[slot], sem_i).wait()
        pltpu.make_async_copy(ew_hbm.at[0], ewbs[slot], sem_i).wait()

    def start_gather(kslot, eslot):
        pltpu.async_copy(tab_hbm.at[ebufs[eslot].at[0]],
                         rows.at[pl.ds(kslot * CK, CK)], sem_g)

    def wait_gather(kslot, eslot):
        pltpu.make_async_copy(tab_hbm.at[ebufs[eslot].at[0]],
                              rows.at[pl.ds(kslot * CK, CK)], sem_g).wait()

    def start_scatter(kslot, eslot):
        pltpu.async_copy(rows.at[pl.ds(kslot * CK, CK)],
                         acc.at[ebufs[eslot].at[1]], sem_s, add=True)

    def wait_scatter(kslot, eslot):
        pltpu.make_async_copy(rows.at[pl.ds(kslot * CK, CK)],
                              acc.at[ebufs[eslot].at[1]], sem_s).wait()

    def scale(kslot, eslot):
        def grp(l16, carry2):
            ew16 = ewbs[eslot][pl.ds(l16 * 16, 16)]
            for i in range(16):
                sc = ew16[i]
                e = kslot * CK + l16 * 16 + i
                for f in range(nsub):
                    v = rows[e, pl.ds(f * 16, 16)]
                    rows[e, pl.ds(f * 16, 16)] = v * sc
            return carry2
        lax.fori_loop(0, CK // 16, grp, 0)

    for j in range(4):
        start_idx(j, w + NW * j)
    wait_idx(0)
    start_gather(0, 0)
    wait_idx(1)
    start_gather(1, 1)

    def it_body(it, carry):
        for u in range(UNROLL):
            if u >= 6:
                pl.when(it < niter - 1)(lambda: wait_idx((u + 2) % NEB))
            else:
                wait_idx((u + 2) % NEB)
            if u < 2:
                pl.when(it > 0)(
                    lambda: wait_scatter((u + 2) % NSLOT, (u + 2) % NEB))
            else:
                wait_scatter((u + 2) % NSLOT, (u + 2) % NEB)
            if u >= 6:
                pl.when(it < niter - 1)(
                    lambda: start_gather((u + 2) % NSLOT, (u + 2) % NEB))
            else:
                start_gather((u + 2) % NSLOT, (u + 2) % NEB)

            wait_gather(u % NSLOT, u)

            ch4 = w + NW * (UNROLL * it + u + 4)
            if u >= 4:
                pl.when(it < niter - 1)(lambda: start_idx((u + 4) % NEB, ch4))
            else:
                start_idx((u + 4) % NEB, ch4)

            scale(u % NSLOT, u)
            start_scatter(u % NSLOT, u)
        return carry

    lax.fori_loop(0, niter, it_body, 0)
    for _ in range(2):
        wait_scatter(0, 0)
    plsc.subcore_barrier()
    pltpu.sync_copy(acc.at[pl.ds(s * slab, slab)],
                    part_hbm.at[c, pl.ds(s * slab, slab)])


def _conv_call(earr, ew2p, table):
    nchunks = earr.shape[0]
    d = table.shape[1]
    n = table.shape[0]
    npad = ((n + 127) // 128) * 128
    body = functools.partial(_conv_body, npad, nchunks, d)
    return pl.kernel(
        body,
        out_type=jax.ShapeDtypeStruct((NC, npad, d), jnp.float32),
        mesh=_sc_mesh(),
        scratch_types=(
            [pltpu.VMEM((2, CK), jnp.int32) for _ in range(NEB)]
            + [pltpu.VMEM((CK,), jnp.float32) for _ in range(NEB)]
            + [
                pltpu.VMEM((NSLOT * CK, d), jnp.float32),
                pltpu.VMEM_SHARED((npad, d), jnp.float32),
                pltpu.SemaphoreType.DMA,
                pltpu.SemaphoreType.DMA,
                pltpu.SemaphoreType.DMA,
            ]
        ),
    )(earr, ew2p, table)


def _dinv(degp_ref, n):
    deg = degp_ref[0, :n] + degp_ref[1, :n] + 1.0
    return lax.rsqrt(deg).reshape(n, 1)


def _mm_body(x_ref, w_ref, out_ref):
    out_ref[...] = jnp.dot(x_ref[...], w_ref[...],
                           preferred_element_type=jnp.float32)


def _prescale_body(n, lin_ref, degp_ref, out_ref):
    dinv = _dinv(degp_ref, n)
    out_ref[...] = dinv * lin_ref[...]


def _mid_body(n, degp_ref, p_ref, linp_ref, w2_ref, b1_ref, out_ref):
    dinv = _dinv(degp_ref, n)
    ssum = p_ref[0, :n, :] + p_ref[1, :n, :] + linp_ref[...]
    h = jnp.maximum(dinv * ssum + b1_ref[...], 0.0)
    out_ref[...] = dinv * jnp.dot(h, w2_ref[...],
                                  preferred_element_type=jnp.float32)


def _final_body(n, degp_ref, p_ref, linp_ref, b2_ref, g_ref, be_ref, x_ref,
                out_ref):
    dinv = _dinv(degp_ref, n)
    out2 = dinv * (p_ref[0, :n, :] + p_ref[1, :n, :] + linp_ref[...]) \
        + b2_ref[...]
    mean = jnp.mean(out2, axis=0)
    var = jnp.mean((out2 - mean) ** 2, axis=0)
    y = g_ref[...] * (out2 - mean) * lax.rsqrt(var + 1e-5) + be_ref[...] \
        + x_ref[...]
    out_ref[...] = jnp.maximum(y, 0.0)


def _tc_call(body, out_shape, *args):
    return pl.pallas_call(
        body, out_shape=jax.ShapeDtypeStruct(out_shape, jnp.float32))(*args)


def kernel(x, edge_index, edge_weight, W1, b1, W2, b2, gamma, beta):
    n, d = x.shape
    e = edge_weight.shape[0]
    npad_deg = ((n + 16 * NS - 1) // (16 * NS)) * (16 * NS)
    assert d % 16 == 0 and n % NS == 0

    grp = CK * NW * UNROLL
    epad = ((e + grp - 1) // grp) * grp
    pad = epad - e
    ar = jnp.arange(pad, dtype=jnp.int32) % n
    row2p = jnp.concatenate([edge_index[0], ar]).reshape(-1, CK)
    col2p = jnp.concatenate([edge_index[1], ar]).reshape(-1, CK)
    ew2p = jnp.concatenate(
        [edge_weight, jnp.zeros((pad,), jnp.float32)]).reshape(-1, CK)
    earr = jnp.stack([row2p, col2p], axis=1)

    lin1 = _tc_call(_mm_body, (n, d), x, W1)
    degp = _deg_call(earr, ew2p, npad_deg)
    lin1p = _tc_call(functools.partial(_prescale_body, n), (n, d),
                     lin1, degp)
    part1 = _conv_call(earr, ew2p, lin1p)
    lin2p = _tc_call(functools.partial(_mid_body, n), (n, d),
                     degp, part1, lin1p, W2, b1)
    part2 = _conv_call(earr, ew2p, lin2p)
    out = _tc_call(functools.partial(_final_body, n), (n, d),
                   degp, part2, lin2p, b2, gamma, beta, x)
    return out


# speedup vs baseline: 1.0213x; 1.0071x over previous
import functools

import jax
import jax.numpy as jnp
from jax import lax
from jax.experimental import pallas as pl
from jax.experimental.pallas import tpu as pltpu
from jax.experimental.pallas import tpu_sc as plsc

NC = 2
NS = 16
NW = NC * NS
CK = 80
NSLOT = 4
NEB = 8
UNROLL = 8


def _sc_mesh():
    return plsc.VectorSubcoreMesh(
        core_axis_name="c", subcore_axis_name="s", num_cores=NC,
        num_subcores=NS)


def _zero_slab(rows, acc, s, slab, nsub, nrow):
    def zr(e, carry):
        for f in range(nsub):
            rows[e, pl.ds(f * 16, 16)] = jnp.zeros((16,), jnp.float32)
        return carry
    lax.fori_loop(0, nrow, zr, 0)
    done = 0
    while done < slab:
        step = min(nrow, slab - done)
        pltpu.sync_copy(rows.at[pl.ds(0, step)],
                        acc.at[pl.ds(s * slab + done, step)])
        done += step


def _deg_body(npad, nchunks, earr_hbm, ew_hbm, degp_hbm, *scr):
    ebufs = scr[0:NEB]
    ewbs = scr[NEB:2 * NEB]
    zbuf, acc, sem_i, sem_s = scr[2 * NEB:]
    slab = npad // NS
    c = lax.axis_index("c")
    s = lax.axis_index("s")
    w = c * NS + s
    nk = nchunks // NW
    niter = nk // UNROLL

    def zb(j, carry):
        zbuf[pl.ds(j * 16, 16)] = jnp.zeros((16,), jnp.float32)
        return carry
    lax.fori_loop(0, (slab + 15) // 16, zb, 0)
    pltpu.sync_copy(zbuf.at[pl.ds(0, slab)], acc.at[pl.ds(s * slab, slab)])
    plsc.subcore_barrier()

    def start_idx(slot, ch):
        pltpu.async_copy(earr_hbm.at[ch], ebufs[slot], sem_i)
        pltpu.async_copy(ew_hbm.at[ch], ewbs[slot], sem_i)

    def wait_idx(slot):
        pltpu.make_async_copy(earr_hbm.at[0], ebufs[slot], sem_i).wait()
        pltpu.make_async_copy(ew_hbm.at[0], ewbs[slot], sem_i).wait()

    def start_scatter(slot):
        pltpu.async_copy(ewbs[slot], acc.at[ebufs[slot].at[1]], sem_s,
                         add=True)

    def wait_scatter(slot):
        pltpu.make_async_copy(ewbs[slot], acc.at[ebufs[slot].at[1]],
                              sem_s).wait()

    for j in range(4):
        start_idx(j, w + NW * j)
    wait_idx(0)
    wait_idx(1)

    def it_body(it, carry):
        for u in range(UNROLL):
            if u >= 6:
                pl.when(it < niter - 1)(lambda: wait_idx((u + 2) % NEB))
            else:
                wait_idx((u + 2) % NEB)
            if u < 2:
                pl.when(it > 0)(lambda: wait_scatter((u + 2) % NEB))
            else:
                wait_scatter((u + 2) % NEB)
            start_scatter(u)
            ch4 = w + NW * (UNROLL * it + u + 4)
            if u >= 4:
                pl.when(it < niter - 1)(lambda: start_idx((u + 4) % NEB, ch4))
            else:
                start_idx((u + 4) % NEB, ch4)
        return carry

    lax.fori_loop(0, niter, it_body, 0)
    for _ in range(2):
        wait_scatter(0)
    plsc.subcore_barrier()
    pltpu.sync_copy(acc.at[pl.ds(s * slab, slab)],
                    degp_hbm.at[c, pl.ds(s * slab, slab)])


def _deg_call(earr, ew2p, npad):
    nchunks = earr.shape[0]
    body = functools.partial(_deg_body, npad, nchunks)
    return pl.kernel(
        body,
        out_type=jax.ShapeDtypeStruct((NC, npad), jnp.float32),
        mesh=_sc_mesh(),
        scratch_types=(
            [pltpu.VMEM((2, CK), jnp.int32) for _ in range(NEB)]
            + [pltpu.VMEM((CK,), jnp.float32) for _ in range(NEB)]
            + [
                pltpu.VMEM((((npad // NS + 15) // 16) * 16,), jnp.float32),
                pltpu.VMEM_SHARED((npad,), jnp.float32),
                pltpu.SemaphoreType.DMA,
                pltpu.SemaphoreType.DMA,
            ]
        ),
    )(earr, ew2p)


def _conv_body(npad, nchunks, d, earr_hbm, ew_hbm, tab_hbm, part_hbm, *scr):
    ebufs = scr[0:NEB]
    ewbs = scr[NEB:2 * NEB]
    rows, acc, sem_i, sem_g, sem_s = scr[2 * NEB:]
    slab = npad // NS
    c = lax.axis_index("c")
    s = lax.axis_index("s")
    w = c * NS + s
    nsub = d // 16
    nk = nchunks // NW
    niter = nk // UNROLL

    _zero_slab(rows, acc, s, slab, nsub, CK)
    plsc.subcore_barrier()

    def start_idx(slot, ch):
        pltpu.async_copy(earr_hbm.at[ch], ebufs[slot], sem_i)
        pltpu.async_copy(ew_hbm.at[ch], ewbs[slot], sem_i)

    def wait_idx(slot):
        pltpu.make_async_copy(earr_hbm.at[0], ebufs[slot], sem_i).wait()
        pltpu.make_async_copy(ew_hbm.at[0], ewbs[slot], sem_i).wait()

    def start_gather(kslot, eslot):
        pltpu.async_copy(tab_hbm.at[ebufs[eslot].at[0]],
                         rows.at[pl.ds(kslot * CK, CK)], sem_g)

    def wait_gather(kslot, eslot):
        pltpu.make_async_copy(tab_hbm.at[ebufs[eslot].at[0]],
                              rows.at[pl.ds(kslot * CK, CK)], sem_g).wait()

    def start_scatter(kslot, eslot):
        pltpu.async_copy(rows.at[pl.ds(kslot * CK, CK)],
                         acc.at[ebufs[eslot].at[1]], sem_s, add=True)

    def wait_scatter(kslot, eslot):
        pltpu.make_async_copy(rows.at[pl.ds(kslot * CK, CK)],
                              acc.at[ebufs[eslot].at[1]], sem_s).wait()

    def scale(kslot, eslot):
        def grp(l16, carry2):
            ew16 = ewbs[eslot][pl.ds(l16 * 16, 16)]
            for i in range(16):
                sc = ew16[i]
                e = kslot * CK + l16 * 16 + i
                for f in range(nsub):
                    v = rows[e, pl.ds(f * 16, 16)]
                    rows[e, pl.ds(f * 16, 16)] = v * sc
            return carry2
        lax.fori_loop(0, CK // 16, grp, 0)

    for j in range(4):
        start_idx(j, w + NW * j)
    wait_idx(0)
    start_gather(0, 0)
    wait_idx(1)
    start_gather(1, 1)

    def it_body(it, carry):
        for u in range(UNROLL):
            if u >= 6:
                pl.when(it < niter - 1)(lambda: wait_idx((u + 2) % NEB))
            else:
                wait_idx((u + 2) % NEB)
            if u < 2:
                pl.when(it > 0)(
                    lambda: wait_scatter((u + 2) % NSLOT, (u + 2) % NEB))
            else:
                wait_scatter((u + 2) % NSLOT, (u + 2) % NEB)
            if u >= 6:
                pl.when(it < niter - 1)(
                    lambda: start_gather((u + 2) % NSLOT, (u + 2) % NEB))
            else:
                start_gather((u + 2) % NSLOT, (u + 2) % NEB)

            wait_gather(u % NSLOT, u)

            ch4 = w + NW * (UNROLL * it + u + 4)
            if u >= 4:
                pl.when(it < niter - 1)(lambda: start_idx((u + 4) % NEB, ch4))
            else:
                start_idx((u + 4) % NEB, ch4)

            scale(u % NSLOT, u)
            start_scatter(u % NSLOT, u)
        return carry

    lax.fori_loop(0, niter, it_body, 0)
    for _ in range(2):
        wait_scatter(0, 0)
    plsc.subcore_barrier()
    pltpu.sync_copy(acc.at[pl.ds(s * slab, slab)],
                    part_hbm.at[c, pl.ds(s * slab, slab)])


def _conv_call(earr, ew2p, table):
    nchunks = earr.shape[0]
    d = table.shape[1]
    n = table.shape[0]
    npad = ((n + 127) // 128) * 128
    body = functools.partial(_conv_body, npad, nchunks, d)
    return pl.kernel(
        body,
        out_type=jax.ShapeDtypeStruct((NC, npad, d), jnp.float32),
        mesh=_sc_mesh(),
        scratch_types=(
            [pltpu.VMEM((2, CK), jnp.int32) for _ in range(NEB)]
            + [pltpu.VMEM((CK,), jnp.float32) for _ in range(NEB)]
            + [
                pltpu.VMEM((NSLOT * CK, d), jnp.float32),
                pltpu.VMEM_SHARED((npad, d), jnp.float32),
                pltpu.SemaphoreType.DMA,
                pltpu.SemaphoreType.DMA,
                pltpu.SemaphoreType.DMA,
            ]
        ),
    )(earr, ew2p, table)


def _dinv(degp_ref, n):
    deg = degp_ref[0, :n] + degp_ref[1, :n] + 1.0
    return lax.rsqrt(deg).reshape(n, 1)


def _prescale_body(n, x_ref, w_ref, degp_ref, out_ref):
    dinv = _dinv(degp_ref, n)
    out_ref[...] = dinv * jnp.dot(x_ref[...], w_ref[...],
                                  preferred_element_type=jnp.float32)


def _mid_body(n, degp_ref, p_ref, linp_ref, w2_ref, b1_ref, out_ref):
    dinv = _dinv(degp_ref, n)
    ssum = p_ref[0, :n, :] + p_ref[1, :n, :] + linp_ref[...]
    h = jnp.maximum(dinv * ssum + b1_ref[...], 0.0)
    out_ref[...] = dinv * jnp.dot(h, w2_ref[...],
                                  preferred_element_type=jnp.float32)


def _final_body(n, degp_ref, p_ref, linp_ref, b2_ref, g_ref, be_ref, x_ref,
                out_ref):
    dinv = _dinv(degp_ref, n)
    out2 = dinv * (p_ref[0, :n, :] + p_ref[1, :n, :] + linp_ref[...]) \
        + b2_ref[...]
    mean = jnp.mean(out2, axis=0)
    var = jnp.mean((out2 - mean) ** 2, axis=0)
    y = g_ref[...] * (out2 - mean) * lax.rsqrt(var + 1e-5) + be_ref[...] \
        + x_ref[...]
    out_ref[...] = jnp.maximum(y, 0.0)


def _tc_call(body, out_shape, *args):
    return pl.pallas_call(
        body, out_shape=jax.ShapeDtypeStruct(out_shape, jnp.float32))(*args)


def kernel(x, edge_index, edge_weight, W1, b1, W2, b2, gamma, beta):
    n, d = x.shape
    e = edge_weight.shape[0]
    npad_deg = ((n + 16 * NS - 1) // (16 * NS)) * (16 * NS)
    assert d % 16 == 0 and n % NS == 0

    grp = CK * NW * UNROLL
    epad = ((e + grp - 1) // grp) * grp
    pad = epad - e
    ar = jnp.arange(pad, dtype=jnp.int32) % n
    row2p = jnp.concatenate([edge_index[0], ar]).reshape(-1, CK)
    col2p = jnp.concatenate([edge_index[1], ar]).reshape(-1, CK)
    ew2p = jnp.concatenate(
        [edge_weight, jnp.zeros((pad,), jnp.float32)]).reshape(-1, CK)
    earr = jnp.stack([row2p, col2p], axis=1)

    degp = _deg_call(earr, ew2p, npad_deg)
    lin1p = _tc_call(functools.partial(_prescale_body, n), (n, d),
                     x, W1, degp)
    part1 = _conv_call(earr, ew2p, lin1p)
    lin2p = _tc_call(functools.partial(_mid_body, n), (n, d),
                     degp, part1, lin1p, W2, b1)
    part2 = _conv_call(earr, ew2p, lin2p)
    out = _tc_call(functools.partial(_final_body, n), (n, d),
                   degp, part2, lin2p, b2, gamma, beta, x)
    return out
